# Initial kernel scaffold; baseline (speedup 1.0000x reference)
import functools

import jax
import jax.numpy as jnp
from jax import lax
from jax.experimental import pallas as pl
from jax.experimental.pallas import tpu as pltpu
from jax.experimental.pallas import tpu_sc as plsc

NUM_CORES = 2
NUM_SUBCORES = 16
NUM_WORKERS = NUM_CORES * NUM_SUBCORES

IDX_VEC = 128
K_PER_STEP = 8
CHUNK = K_PER_STEP * IDX_VEC


def _gather_call(n_rows, dim):
    n_vecs = n_rows // IDX_VEC
    vecs_per_worker = n_vecs // NUM_WORKERS
    steps = vecs_per_worker // K_PER_STEP
    rows_per_worker = n_rows // NUM_WORKERS

    mesh = plsc.VectorSubcoreMesh(core_axis_name="c", subcore_axis_name="s")

    @functools.partial(
        pl.kernel,
        mesh=mesh,
        out_type=jax.ShapeDtypeStruct((n_rows, dim), jnp.float32),
        scratch_types=[
            pltpu.VMEM((K_PER_STEP, IDX_VEC), jnp.int32),
            pltpu.VMEM((CHUNK, dim), jnp.float32),
            pltpu.SemaphoreType.DMA,
        ],
    )
    def k(idx_hbm, table_hbm, out_hbm, idx_v, rows_v, sem):
        wid = lax.axis_index("s") * NUM_CORES + lax.axis_index("c")
        vec_base = wid * vecs_per_worker
        row_base = wid * rows_per_worker

        def body(i, _):
            pltpu.sync_copy(
                idx_hbm.at[pl.ds(vec_base + i * K_PER_STEP, K_PER_STEP)],
                idx_v,
            )
            copies = []
            for j in range(K_PER_STEP):
                copies.append(
                    pltpu.async_copy(
                        table_hbm.at[idx_v.at[j]],
                        rows_v.at[pl.ds(j * IDX_VEC, IDX_VEC)],
                        sem,
                    )
                )
            for c in copies:
                c.wait()
            pltpu.sync_copy(
                rows_v,
                out_hbm.at[pl.ds(row_base + i * CHUNK, CHUNK)],
            )
            return 0

        lax.fori_loop(0, steps, body, 0)

    return k


def kernel(token_ids, weight):
    b, s = token_ids.shape
    dim = weight.shape[1]
    n_rows = b * s
    ids = token_ids.astype(jnp.int32).reshape(n_rows // IDX_VEC, IDX_VEC)
    out = _gather_call(n_rows, dim)(ids, weight)
    return out.reshape(b, s, dim)


# speedup vs baseline: 1.0946x; 1.0946x over previous
import functools

import jax
import jax.numpy as jnp
from jax import lax
from jax.experimental import pallas as pl
from jax.experimental.pallas import tpu as pltpu
from jax.experimental.pallas import tpu_sc as plsc

NUM_CORES = 2
NUM_SUBCORES = 16
NUM_WORKERS = NUM_CORES * NUM_SUBCORES

IDX_VEC = 128
K_PER_STEP = 8
CHUNK = K_PER_STEP * IDX_VEC


def _gather_call(n_rows, dim):
    n_vecs = n_rows // IDX_VEC
    vecs_per_worker = n_vecs // NUM_WORKERS
    steps = vecs_per_worker // K_PER_STEP
    rows_per_worker = n_rows // NUM_WORKERS

    mesh = plsc.VectorSubcoreMesh(core_axis_name="c", subcore_axis_name="s")

    @functools.partial(
        pl.kernel,
        mesh=mesh,
        out_type=jax.ShapeDtypeStruct((n_rows, dim), jnp.float32),
        scratch_types=[
            pltpu.VMEM((K_PER_STEP, IDX_VEC), jnp.int32),
            pltpu.VMEM((CHUNK, dim), jnp.float32),
            pltpu.SemaphoreType.DMA,
        ],
        compiler_params=pltpu.CompilerParams(use_tc_tiling_on_sc=False),
    )
    def k(idx_hbm, table_hbm, out_hbm, idx_v, rows_v, sem):
        wid = lax.axis_index("s") * NUM_CORES + lax.axis_index("c")
        vec_base = wid * vecs_per_worker
        row_base = wid * rows_per_worker

        def body(i, _):
            pltpu.sync_copy(
                idx_hbm.at[pl.ds(vec_base + i * K_PER_STEP, K_PER_STEP)],
                idx_v,
            )
            copies = []
            for j in range(K_PER_STEP):
                copies.append(
                    pltpu.async_copy(
                        table_hbm.at[idx_v.at[j]],
                        rows_v.at[pl.ds(j * IDX_VEC, IDX_VEC)],
                        sem,
                    )
                )
            for c in copies:
                c.wait()
            pltpu.sync_copy(
                rows_v,
                out_hbm.at[pl.ds(row_base + i * CHUNK, CHUNK)],
            )
            return 0

        lax.fori_loop(0, steps, body, 0)

    return k


def kernel(token_ids, weight):
    b, s = token_ids.shape
    dim = weight.shape[1]
    n_rows = b * s
    ids = token_ids.astype(jnp.int32).reshape(n_rows // IDX_VEC, IDX_VEC)
    out = _gather_call(n_rows, dim)(ids, weight)
    return out.reshape(b, s, dim)


# speedup vs baseline: 1.1066x; 1.0110x over previous
import functools

import jax
import jax.numpy as jnp
from jax import lax
from jax.experimental import pallas as pl
from jax.experimental.pallas import tpu as pltpu
from jax.experimental.pallas import tpu_sc as plsc

NUM_CORES = 2
NUM_SUBCORES = 16
NUM_WORKERS = NUM_CORES * NUM_SUBCORES

IDX_VEC = 128
K_PER_STEP = 8
CHUNK = K_PER_STEP * IDX_VEC


def _gather_call(n_rows, dim):
    n_vecs = n_rows // IDX_VEC
    vecs_per_worker = n_vecs // NUM_WORKERS
    steps = vecs_per_worker // K_PER_STEP
    rows_per_worker = n_rows // NUM_WORKERS
    assert steps % 2 == 1 and steps >= 3

    mesh = plsc.VectorSubcoreMesh(core_axis_name="c", subcore_axis_name="s")

    @functools.partial(
        pl.kernel,
        mesh=mesh,
        out_type=jax.ShapeDtypeStruct((n_rows, dim), jnp.float32),
        scratch_types=[
            pltpu.VMEM((2, K_PER_STEP, IDX_VEC), jnp.int32),
            pltpu.VMEM((2, CHUNK, dim), jnp.float32),
            pltpu.SemaphoreType.DMA,
            pltpu.SemaphoreType.DMA,
            pltpu.SemaphoreType.DMA,
            pltpu.SemaphoreType.DMA,
        ],
        compiler_params=pltpu.CompilerParams(use_tc_tiling_on_sc=False),
    )
    def k(idx_hbm, table_hbm, out_hbm, idx_v, rows_v, g0, g1, o0, o1):
        gsem = (g0, g1)
        osem = (o0, o1)
        wid = lax.axis_index("s") * NUM_CORES + lax.axis_index("c")
        vec_base = wid * vecs_per_worker
        row_base = wid * rows_per_worker

        def load_and_fire(g, b):
            pltpu.sync_copy(
                idx_hbm.at[pl.ds(vec_base + g * K_PER_STEP, K_PER_STEP)],
                idx_v.at[b],
            )
            for j in range(K_PER_STEP):
                pltpu.async_copy(
                    table_hbm.at[idx_v.at[b].at[j]],
                    rows_v.at[b].at[pl.ds(j * IDX_VEC, IDX_VEC)],
                    gsem[b],
                )

        def wait_gathers(b):
            for j in range(K_PER_STEP):
                pltpu.make_async_copy(
                    table_hbm.at[idx_v.at[b].at[j]],
                    rows_v.at[b].at[pl.ds(j * IDX_VEC, IDX_VEC)],
                    gsem[b],
                ).wait()

        def fire_writeback(g, b):
            pltpu.async_copy(
                rows_v.at[b],
                out_hbm.at[pl.ds(row_base + g * CHUNK, CHUNK)],
                osem[b],
            )

        def wait_writeback(g, b):
            pltpu.make_async_copy(
                rows_v.at[b],
                out_hbm.at[pl.ds(row_base + g * CHUNK, CHUNK)],
                osem[b],
            ).wait()

        load_and_fire(0, 0)

        def body(t, _):
            ga = 2 * t + 1

            @pl.when(t >= 1)
            def _():
                wait_writeback(ga - 2, 1)

            load_and_fire(ga, 1)
            wait_gathers(0)
            fire_writeback(ga - 1, 0)

            gb = ga + 1
            wait_writeback(gb - 2, 0)
            load_and_fire(gb, 0)
            wait_gathers(1)
            fire_writeback(gb - 1, 1)
            return 0

        lax.fori_loop(0, (steps - 1) // 2, body, 0)

        last = steps - 1
        wait_gathers(0)
        fire_writeback(last, 0)
        wait_writeback(last - 1, 1)
        wait_writeback(last, 0)

    return k


def kernel(token_ids, weight):
    b, s = token_ids.shape
    dim = weight.shape[1]
    n_rows = b * s
    ids = token_ids.astype(jnp.int32).reshape(n_rows // IDX_VEC, IDX_VEC)
    out = _gather_call(n_rows, dim)(ids, weight)
    return out.reshape(b, s, dim)
